# baseline (speedup 1.0000x reference)
import functools

import jax
import jax.numpy as jnp
from jax import lax
from jax.experimental import pallas as pl
from jax.experimental.pallas import tpu as pltpu
from jax.experimental.pallas import tpu_sc as plsc

B, L = 16384, 200
N = B * L
V, D = 800, 128
NC, NS = 2, 16
NW = NC * NS
LANES = 16
CHUNK = 25600
PER_W = N // NW
TRIPS = PER_W // CHUNK

ROWS = 64
ROW_TILES = 13


def _sc_body(x_hbm, table_hbm, w_hbm, b_hbm, out_hbm,
             tab_v, w_v, b_v, q_local, q_shared, q_v, idx_v, out_v):
    cid = lax.axis_index("c")
    sid = lax.axis_index("s")

    row0 = jnp.minimum(sid * ROWS, V - ROWS)
    pltpu.sync_copy(w_hbm, w_v)
    pltpu.sync_copy(b_hbm, b_v)

    @pl.when(sid < ROW_TILES)
    def _compute_rows():
        pltpu.sync_copy(table_hbm.at[pl.ds(row0, ROWS)], tab_v)
        bias = b_v[pl.ds(0, LANES)][0]
        lane = lax.broadcasted_iota(jnp.int32, (LANES,), 0)
        wd = [w_v[pl.ds((d // LANES) * LANES, LANES)][d % LANES] for d in range(D)]
        for g in range(ROWS // LANES):
            rows = lane + g * LANES
            acc = jnp.zeros((LANES,), jnp.float32)
            for d in range(D):
                col = plsc.load_gather(tab_v, [rows, jnp.full((LANES,), d, jnp.int32)])
                acc = acc + col * wd[d]
            q_local[pl.ds(g * LANES, LANES)] = jnp.maximum(acc + bias, 0.0)
        pltpu.sync_copy(q_local, q_shared.at[pl.ds(row0, ROWS)])

    plsc.subcore_barrier()
    pltpu.sync_copy(q_shared, q_v)

    w = cid * NS + sid
    base = w * PER_W
    for t in range(TRIPS):
        off = base + t * CHUNK
        pltpu.sync_copy(x_hbm.at[pl.ds(off, CHUNK)], idx_v)

        def _inner(m, carry):
            mb = m * (8 * LANES)
            for jb in range(8):
                iv = idx_v[pl.ds(mb + jb * LANES, LANES)]
                out_v[pl.ds(mb + jb * LANES, LANES)] = plsc.load_gather(q_v, [iv])
            return carry

        lax.fori_loop(0, CHUNK // (8 * LANES), _inner, 0)
        pltpu.sync_copy(out_v, out_hbm.at[pl.ds(off, CHUNK)])


_sc_gather = functools.partial(
    pl.kernel,
    out_type=jax.ShapeDtypeStruct((N,), jnp.float32),
    mesh=plsc.VectorSubcoreMesh(core_axis_name="c", subcore_axis_name="s"),
    compiler_params=pltpu.CompilerParams(needs_layout_passes=False),
    scratch_types=[
        pltpu.VMEM((ROWS, D), jnp.float32),
        pltpu.VMEM((D,), jnp.float32),
        pltpu.VMEM((LANES,), jnp.float32),
        pltpu.VMEM((ROWS,), jnp.float32),
        pltpu.VMEM_SHARED((V,), jnp.float32),
        pltpu.VMEM((V,), jnp.float32),
        pltpu.VMEM((CHUNK,), jnp.int32),
        pltpu.VMEM((CHUNK,), jnp.float32),
    ],
)(_sc_body)


def kernel(x1, table, W, b):
    x_flat = x1.astype(jnp.int32).reshape(-1)
    w_vec = W.reshape(-1)
    b_vec = jnp.broadcast_to(b.reshape(-1), (LANES,))
    out_flat = _sc_gather(x_flat, table, w_vec, b_vec)
    return out_flat.reshape(B, L, 1)


# speedup vs baseline: 1.9150x; 1.9150x over previous
import functools

import jax
import jax.numpy as jnp
from jax import lax
from jax.experimental import pallas as pl
from jax.experimental.pallas import tpu as pltpu
from jax.experimental.pallas import tpu_sc as plsc

B, L = 16384, 200
N = B * L
V, D = 800, 128
NC, NS = 2, 16
NW = NC * NS
LANES = 16
MAX_T = (L + NW - 1) // NW

ROWS = 64
ROW_TILES = 13


def _sc_body(xt_hbm, table_hbm, w_hbm, b_hbm, out_hbm,
             tab_v, w_v, b_v, q_local, q_shared, q_v, idx_v, out_v):
    cid = lax.axis_index("c")
    sid = lax.axis_index("s")

    row0 = jnp.minimum(sid * ROWS, V - ROWS)
    pltpu.sync_copy(w_hbm, w_v)
    pltpu.sync_copy(b_hbm, b_v)

    @pl.when(sid < ROW_TILES)
    def _compute_rows():
        pltpu.sync_copy(table_hbm.at[pl.ds(row0, ROWS)], tab_v)
        bias = b_v[pl.ds(0, LANES)][0]
        lane = lax.broadcasted_iota(jnp.int32, (LANES,), 0)
        wd = [w_v[pl.ds((d // LANES) * LANES, LANES)][d % LANES] for d in range(D)]
        for g in range(ROWS // LANES):
            rows = lane + g * LANES
            acc = jnp.zeros((LANES,), jnp.float32)
            for d in range(D):
                col = plsc.load_gather(tab_v, [rows, jnp.full((LANES,), d, jnp.int32)])
                acc = acc + col * wd[d]
            q_local[pl.ds(g * LANES, LANES)] = jnp.maximum(acc + bias, 0.0)
        pltpu.sync_copy(q_local, q_shared.at[pl.ds(row0, ROWS)])

    plsc.subcore_barrier()
    pltpu.sync_copy(q_shared, q_v)

    w = cid * NS + sid
    for t in range(MAX_T):
        r = w + t * NW

        @pl.when(r < L)
        def _row():
            pltpu.sync_copy(xt_hbm.at[r], idx_v)

            def _inner(m, carry):
                mb = m * (8 * LANES)
                for jb in range(8):
                    iv = idx_v[pl.ds(mb + jb * LANES, LANES)]
                    out_v[pl.ds(mb + jb * LANES, LANES)] = plsc.load_gather(q_v, [iv])
                return carry

            lax.fori_loop(0, B // (8 * LANES), _inner, 0)
            pltpu.sync_copy(out_v, out_hbm.at[pl.ds(r * B, B)])


_sc_gather = functools.partial(
    pl.kernel,
    out_type=jax.ShapeDtypeStruct((N,), jnp.float32),
    mesh=plsc.VectorSubcoreMesh(core_axis_name="c", subcore_axis_name="s"),
    compiler_params=pltpu.CompilerParams(needs_layout_passes=False),
    scratch_types=[
        pltpu.VMEM((ROWS, D), jnp.float32),
        pltpu.VMEM((D,), jnp.float32),
        pltpu.VMEM((LANES,), jnp.float32),
        pltpu.VMEM((ROWS,), jnp.float32),
        pltpu.VMEM_SHARED((V,), jnp.float32),
        pltpu.VMEM((V,), jnp.float32),
        pltpu.VMEM((B,), jnp.int32),
        pltpu.VMEM((B,), jnp.float32),
    ],
)(_sc_body)


def kernel(x1, table, W, b):
    xt = jnp.swapaxes(x1.astype(jnp.int32), 0, 1)
    w_vec = W.reshape(-1)
    b_vec = jnp.broadcast_to(b.reshape(-1), (LANES,))
    out_flat = _sc_gather(xt, table, w_vec, b_vec)
    return jnp.transpose(out_flat.reshape(L, B, 1), (1, 0, 2))
